# baseline (speedup 1.0000x reference)
import functools

import jax
import jax.numpy as jnp
from jax import lax
from jax.experimental import pallas as pl
from jax.experimental.pallas import tpu as pltpu
from jax.experimental.pallas import tpu_sc as plsc

N = 50000
E = 800000
EMB = 64
QF = 16
NQ = 4
NSUB = 16
NCORE = 2
CHUNK = 128
GROUP = 8
ITERS = 50
HITERS = ITERS // 2
EROWS = NSUB * ITERS * GROUP
E_PAD = EROWS * CHUNK
SUB_EROWS = ITERS * GROUP

NACC = 50048
DUMMY = 50040
ROWS_PER_SUB = NACC // NSUB

NROWS_R = NQ * NACC * QF // 128
BLK_R = 1088
GRID_R = NROWS_R // BLK_R


def _sc_mesh():
    return plsc.VectorSubcoreMesh(core_axis_name="c", subcore_axis_name="s")


def _edge_pass_body(with_gather, *refs):
    if with_gather:
        (row3, col3, y4, zeros, out,
         idx_r_v, idx_c_v, rows_v, acc, sem_g, sem_s0, sem_s1) = refs
    else:
        (col3, zeros, ones, out,
         idx_r_v, idx_c_v, rows_v, acc, sem_g, sem_s0, sem_s1) = refs
    c = lax.axis_index("c")
    s = lax.axis_index("s")
    sem_s = (sem_s0, sem_s1)

    def base(g):
        return s * SUB_EROWS + g * GROUP

    def load_r(slot, g):
        pltpu.sync_copy(row3.at[pl.ds(base(g), GROUP)], idx_r_v.at[slot])

    def load_c(slot, g):
        pltpu.sync_copy(col3.at[pl.ds(base(g), GROUP)], idx_c_v.at[slot])

    def gath(q, slot):
        for j in range(GROUP):
            pltpu.async_copy(
                y4.at[q].at[idx_r_v.at[slot].at[j]], rows_v.at[slot].at[j],
                sem_g)

    def wait_gath(q, slot):
        for j in range(GROUP):
            pltpu.make_async_copy(
                y4.at[q].at[idx_r_v.at[slot].at[j]], rows_v.at[slot].at[j],
                sem_g).wait()

    def scat(src_slot, slot):
        for j in range(GROUP):
            pltpu.async_copy(
                rows_v.at[src_slot].at[j], acc.at[idx_c_v.at[slot].at[j]],
                sem_s[slot], add=True)

    def wait_scat(src_slot, slot):
        for j in range(GROUP):
            pltpu.make_async_copy(
                rows_v.at[src_slot].at[j], acc.at[idx_c_v.at[slot].at[j]],
                sem_s[slot]).wait()

    if not with_gather:
        for j in range(GROUP):
            pltpu.sync_copy(ones, rows_v.at[0].at[j])

    n_passes = 2 if with_gather else 1
    for p in range(n_passes):
        q = 2 * c + p
        pltpu.sync_copy(zeros, acc.at[pl.ds(s * ROWS_PER_SUB, ROWS_PER_SUB)])
        plsc.subcore_barrier()

        if with_gather:
            def half_iter(h, first, last):
                g0 = 2 * h
                g1 = g0 + 1
                wait_gath(q, 0)
                load_r(1, g1)
                if not first:
                    wait_scat(1, 1)
                load_c(0, g0)
                scat(0, 0)
                gath(q, 1)
                wait_gath(q, 1)
                if not last:
                    load_r(0, g0 + 2)
                wait_scat(0, 0)
                load_c(1, g1)
                scat(1, 1)
                if not last:
                    gath(q, 0)

            load_r(0, 0)
            gath(q, 0)
            half_iter(0, True, False)
            lax.fori_loop(
                1, HITERS - 1,
                lambda h, car: (half_iter(h, False, False), car)[1], 0)
            half_iter(HITERS - 1, False, True)
            wait_scat(1, 1)
        else:
            def deg_half_iter(h, first):
                g0 = 2 * h
                g1 = g0 + 1
                if not first:
                    wait_scat(0, 0)
                load_c(0, g0)
                scat(0, 0)
                if not first:
                    wait_scat(0, 1)
                load_c(1, g1)
                scat(0, 1)

            deg_half_iter(0, True)
            lax.fori_loop(
                1, HITERS,
                lambda h, car: (deg_half_iter(h, False), car)[1], 0)
            wait_scat(0, 0)
            wait_scat(0, 1)

        plsc.subcore_barrier()
        sl = pl.ds(s * ROWS_PER_SUB, ROWS_PER_SUB)
        if with_gather:
            pltpu.sync_copy(acc.at[sl], out.at[q].at[sl])
        else:
            pltpu.sync_copy(acc.at[sl], out.at[2 * c].at[sl])
            pltpu.sync_copy(acc.at[sl], out.at[2 * c + 1].at[sl])
        plsc.subcore_barrier()


def _make_edge_pass(with_gather):
    return pl.kernel(
        functools.partial(_edge_pass_body, with_gather),
        out_type=jax.ShapeDtypeStruct((NQ, NACC, QF), jnp.float32),
        mesh=_sc_mesh(),
        compiler_params=pltpu.CompilerParams(use_tc_tiling_on_sc=False),
        scratch_types=[
            pltpu.VMEM((2, GROUP, CHUNK), jnp.int32),
            pltpu.VMEM((2, GROUP, CHUNK), jnp.int32),
            pltpu.VMEM((2, GROUP, CHUNK, QF), jnp.float32),
            pltpu.VMEM_SHARED((NACC, QF), jnp.float32),
            pltpu.SemaphoreType.DMA,
            pltpu.SemaphoreType.DMA,
            pltpu.SemaphoreType.DMA,
        ],
    )


_sc_scatter = _make_edge_pass(True)
_sc_deg = _make_edge_pass(False)


def _tc_prep_body(deg_ref, x_ref, dis_ref, y_ref):
    deg = deg_ref[...]
    dis = jnp.where(deg > 0.0, lax.rsqrt(deg), 0.0)
    dis_ref[...] = dis
    y_ref[...] = dis * x_ref[...]


def _tc_prep(deg_r, x_r):
    spec = pl.BlockSpec((BLK_R, 128), lambda i: (i, 0))
    return pl.pallas_call(
        _tc_prep_body,
        grid=(GRID_R,),
        in_specs=[spec, spec],
        out_specs=[spec, spec],
        out_shape=[jax.ShapeDtypeStruct((NROWS_R, 128), jnp.float32)] * 2,
    )(deg_r, x_r)


def _tc_scale_body(is_final, z_ref, dis_ref, s_ref, so_ref, y_ref):
    dis = dis_ref[...]
    xk = dis * z_ref[...]
    snew = s_ref[...] + xk
    so_ref[...] = snew * 0.25 if is_final else snew
    y_ref[...] = dis * xk


def _tc_scale(z_r, dis_r, s_r, is_final):
    spec = pl.BlockSpec((BLK_R, 128), lambda i: (i, 0))
    return pl.pallas_call(
        functools.partial(_tc_scale_body, is_final),
        grid=(GRID_R,),
        in_specs=[spec, spec, spec],
        out_specs=[spec, spec],
        out_shape=[jax.ShapeDtypeStruct((NROWS_R, 128), jnp.float32)] * 2,
    )(z_r, dis_r, s_r)


def kernel(edge_index, user_emb, item_emb):
    x0 = jnp.concatenate([user_emb, item_emb], axis=0)
    x0p = jnp.pad(x0, ((0, NACC - N), (0, 0)))
    x0s = jnp.stack([x0p[:, q * QF:(q + 1) * QF] for q in range(NQ)])
    x0_r = x0s.reshape(NROWS_R, 128)
    pad_e = E_PAD - E
    row3 = jnp.concatenate(
        [edge_index[0], jnp.zeros((pad_e,), jnp.int32)]).reshape(EROWS, CHUNK)
    col3 = jnp.concatenate(
        [edge_index[1], jnp.full((pad_e,), DUMMY, jnp.int32)]).reshape(EROWS, CHUNK)
    zeros = jnp.zeros((ROWS_PER_SUB, QF), jnp.float32)
    ones = jnp.ones((CHUNK, QF), jnp.float32)

    deg4 = _sc_deg(col3, zeros, ones)
    dis_r, y_r = _tc_prep(deg4.reshape(NROWS_R, 128), x0_r)
    s_r = x0_r
    for k in range(3):
        z4 = _sc_scatter(row3, col3, y_r.reshape(NQ, NACC, QF), zeros)
        s_r, y_r = _tc_scale(z4.reshape(NROWS_R, 128), dis_r, s_r, k == 2)

    f4 = s_r.reshape(NQ, NACC, QF)[:, :N, :]
    final = jnp.concatenate([f4[q] for q in range(NQ)], axis=1)
    return final[:N // 2], final[N // 2:]


# speedup vs baseline: 1.4502x; 1.4502x over previous
import functools

import jax
import jax.numpy as jnp
from jax import lax
from jax.experimental import pallas as pl
from jax.experimental.pallas import tpu as pltpu
from jax.experimental.pallas import tpu_sc as plsc

NU = 25000
PADB = 25024
NACC = 2 * PADB
QF = 16
NQ = 4
NSUB = 16
NCORE = 2
CHUNK = 128
GROUP = 8
ITERS = 49
EROWS = NSUB * ITERS * GROUP
E = 800000
E_PAD = EROWS * CHUNK
SUB_EROWS = ITERS * GROUP
DUMMY = 25016
ROWS_PER_SUB = NACC // NSUB
BOUND_REAL = 3104
WB = ((0, 1000), (1000, 1000), (2000, 1000), (3000, 128))


def _sc_mesh():
    return plsc.VectorSubcoreMesh(core_axis_name="c", subcore_axis_name="s")


def _rsqrt16(d):
    i = lax.bitcast_convert_type(d, jnp.int32)
    i = jnp.int32(0x5F3759DF) - lax.shift_right_arithmetic(i, 1)
    y = lax.bitcast_convert_type(i, jnp.float32)
    for _ in range(3):
        y = y * (1.5 - 0.5 * d * y * y)
    return jnp.where(d > 0.0, y, 0.0)


def _zero_acc(s, zeros, acc):
    pltpu.sync_copy(zeros, acc.at[pl.ds(s * ROWS_PER_SUB, ROWS_PER_SUB)])


def _x_read(s, off, sz, user, item, zeros, xbuf, col_off):
    base = s * ROWS_PER_SUB + off
    csl = pl.ds(col_off, QF)
    if off < 3000:
        @pl.when(s < 8)
        def _():
            pltpu.sync_copy(user.at[pl.ds(base, sz), csl],
                            xbuf.at[pl.ds(0, sz)])

        @pl.when(s >= 8)
        def _():
            pltpu.sync_copy(item.at[pl.ds(base - PADB, sz), csl],
                            xbuf.at[pl.ds(0, sz)])
    else:
        clip = BOUND_REAL - off
        @pl.when(s < 7)
        def _():
            pltpu.sync_copy(user.at[pl.ds(base, sz), csl],
                            xbuf.at[pl.ds(0, sz)])

        @pl.when(s == 7)
        def _():
            pltpu.sync_copy(user.at[pl.ds(base, clip), csl],
                            xbuf.at[pl.ds(0, clip)])
            pltpu.sync_copy(zeros.at[pl.ds(0, sz - clip)],
                            xbuf.at[pl.ds(clip, sz - clip)])

        @pl.when((s >= 8) & (s < 15))
        def _():
            pltpu.sync_copy(item.at[pl.ds(base - PADB, sz), csl],
                            xbuf.at[pl.ds(0, sz)])

        @pl.when(s == 15)
        def _():
            pltpu.sync_copy(item.at[pl.ds(base - PADB, clip), csl],
                            xbuf.at[pl.ds(0, clip)])
            pltpu.sync_copy(zeros.at[pl.ds(0, sz - clip)],
                            xbuf.at[pl.ds(clip, sz - clip)])


def _out_write(s, off, sz, user_o, item_o, sbuf, col_off):
    base = s * ROWS_PER_SUB + off
    csl = pl.ds(col_off, QF)
    if off < 3000:
        @pl.when(s < 8)
        def _():
            pltpu.sync_copy(sbuf.at[pl.ds(0, sz)],
                            user_o.at[pl.ds(base, sz), csl])

        @pl.when(s >= 8)
        def _():
            pltpu.sync_copy(sbuf.at[pl.ds(0, sz)],
                            item_o.at[pl.ds(base - PADB, sz), csl])
    else:
        clip = BOUND_REAL - off
        @pl.when(s < 7)
        def _():
            pltpu.sync_copy(sbuf.at[pl.ds(0, sz)],
                            user_o.at[pl.ds(base, sz), csl])

        @pl.when(s == 7)
        def _():
            pltpu.sync_copy(sbuf.at[pl.ds(0, clip)],
                            user_o.at[pl.ds(base, clip), csl])

        @pl.when((s >= 8) & (s < 15))
        def _():
            pltpu.sync_copy(sbuf.at[pl.ds(0, sz)],
                            item_o.at[pl.ds(base - PADB, sz), csl])

        @pl.when(s == 15)
        def _():
            pltpu.sync_copy(sbuf.at[pl.ds(0, clip)],
                            item_o.at[pl.ds(base - PADB, clip), csl])


def _k1_body(col3, user, item, zeros, ones, dis_o, y_o, s_o,
             idx_c, ones_v, xbuf, dbuf, acc, sem_i, sem_s):
    c = lax.axis_index("c")
    s = lax.axis_index("s")
    pltpu.sync_copy(ones, ones_v)
    _zero_acc(s, zeros, acc)
    plsc.subcore_barrier()

    def deg_iter(g, car):
        base = s * SUB_EROWS + g * GROUP
        pltpu.async_copy(col3.at[pl.ds(base, GROUP)], idx_c, sem_i).wait()
        scs = [pltpu.async_copy(ones_v, acc.at[idx_c.at[j]], sem_s, add=True)
               for j in range(GROUP)]
        for x in scs:
            x.wait()
        return car

    lax.fori_loop(0, ITERS, deg_iter, 0)
    plsc.subcore_barrier()

    for off, sz in WB:
        absr = s * ROWS_PER_SUB + off
        pltpu.sync_copy(acc.at[pl.ds(absr, sz)], dbuf.at[pl.ds(0, sz)])

        def dis_row(i, car):
            for u in range(4):
                dbuf[i * 4 + u] = _rsqrt16(dbuf[i * 4 + u])
            return car

        lax.fori_loop(0, sz // 4, dis_row, 0)

        @pl.when(c == 0)
        def _():
            pltpu.sync_copy(dbuf.at[pl.ds(0, sz)], dis_o.at[pl.ds(absr, sz)])

        for p in range(2):
            q = 2 * c + p
            col_off = pl.multiple_of(q * QF, QF)
            _x_read(s, off, sz, user, item, zeros, xbuf, col_off)
            pltpu.sync_copy(xbuf.at[pl.ds(0, sz)],
                            s_o.at[q].at[pl.ds(absr, sz)])

            def y0_row(i, car):
                for u in range(4):
                    xbuf[i * 4 + u] = dbuf[i * 4 + u] * xbuf[i * 4 + u]
                return car

            lax.fori_loop(0, sz // 4, y0_row, 0)
            pltpu.sync_copy(xbuf.at[pl.ds(0, sz)],
                            y_o.at[q].at[pl.ds(absr, sz)])


def _k2_body(is_final, *refs):
    (row3, col3, y4, dis, s_in, zeros, out_a, out_b,
     idx_r, idx_c, rows_v, disb, sbuf, dbuf, acc,
     sem_i, sem_g, sem_s) = refs
    c = lax.axis_index("c")
    s = lax.axis_index("s")

    for p in range(2):
        q = 2 * c + p
        _zero_acc(s, zeros, acc)
        plsc.subcore_barrier()

        def edge_iter(g, car):
            base = s * SUB_EROWS + g * GROUP
            cp_r = pltpu.async_copy(row3.at[pl.ds(base, GROUP)], idx_r, sem_i)
            cp_c = pltpu.async_copy(col3.at[pl.ds(base, GROUP)], idx_c, sem_i)
            cp_r.wait()
            cp_c.wait()
            gs = [pltpu.async_copy(y4.at[q].at[idx_r.at[j]], rows_v.at[j],
                                   sem_g)
                  for j in range(GROUP)]
            for x in gs:
                x.wait()
            scs = [pltpu.async_copy(rows_v.at[j], acc.at[idx_c.at[j]], sem_s,
                                    add=True)
                   for j in range(GROUP)]
            for x in scs:
                x.wait()
            return car

        lax.fori_loop(0, ITERS, edge_iter, 0)
        plsc.subcore_barrier()

        for off, sz in WB:
            absr = s * ROWS_PER_SUB + off
            pltpu.sync_copy(acc.at[pl.ds(absr, sz)], dbuf.at[pl.ds(0, sz)])
            pltpu.sync_copy(dis.at[pl.ds(absr, sz)], disb.at[pl.ds(0, sz)])
            pltpu.sync_copy(s_in.at[q].at[pl.ds(absr, sz)],
                            sbuf.at[pl.ds(0, sz)])

            def wb_row(i, car):
                for u in range(4):
                    k = i * 4 + u
                    di = disb[k]
                    x = di * dbuf[k]
                    sn = sbuf[k] + x
                    if is_final:
                        sbuf[k] = sn * 0.25
                    else:
                        sbuf[k] = sn
                        dbuf[k] = di * x
                return car

            lax.fori_loop(0, sz // 4, wb_row, 0)

            if is_final:
                col_off = pl.multiple_of(q * QF, QF)
                _out_write(s, off, sz, out_a, out_b, sbuf, col_off)
            else:
                pltpu.sync_copy(sbuf.at[pl.ds(0, sz)],
                                out_b.at[q].at[pl.ds(absr, sz)])
                pltpu.sync_copy(dbuf.at[pl.ds(0, sz)],
                                out_a.at[q].at[pl.ds(absr, sz)])
        plsc.subcore_barrier()


_QSHAPE = jax.ShapeDtypeStruct((NQ, NACC, QF), jnp.float32)

_k1 = pl.kernel(
    _k1_body,
    out_type=(
        jax.ShapeDtypeStruct((NACC, QF), jnp.float32),
        _QSHAPE,
        _QSHAPE,
    ),
    mesh=_sc_mesh(),
    compiler_params=pltpu.CompilerParams(use_tc_tiling_on_sc=False),
    scratch_types=[
        pltpu.VMEM((GROUP, CHUNK), jnp.int32),
        pltpu.VMEM((CHUNK, QF), jnp.float32),
        pltpu.VMEM((1000, QF), jnp.float32),
        pltpu.VMEM((1000, QF), jnp.float32),
        pltpu.VMEM_SHARED((NACC, QF), jnp.float32),
        pltpu.SemaphoreType.DMA,
        pltpu.SemaphoreType.DMA,
    ],
)


def _make_k2(is_final):
    if is_final:
        out_type = (
            jax.ShapeDtypeStruct((NU, 4 * QF), jnp.float32),
            jax.ShapeDtypeStruct((NU, 4 * QF), jnp.float32),
        )
    else:
        out_type = (_QSHAPE, _QSHAPE)
    return pl.kernel(
        functools.partial(_k2_body, is_final),
        out_type=out_type,
        mesh=_sc_mesh(),
        compiler_params=pltpu.CompilerParams(use_tc_tiling_on_sc=False),
        scratch_types=[
            pltpu.VMEM((GROUP, CHUNK), jnp.int32),
            pltpu.VMEM((GROUP, CHUNK), jnp.int32),
            pltpu.VMEM((GROUP, CHUNK, QF), jnp.float32),
            pltpu.VMEM((1000, QF), jnp.float32),
            pltpu.VMEM((1000, QF), jnp.float32),
            pltpu.VMEM((1000, QF), jnp.float32),
            pltpu.VMEM_SHARED((NACC, QF), jnp.float32),
            pltpu.SemaphoreType.DMA,
            pltpu.SemaphoreType.DMA,
            pltpu.SemaphoreType.DMA,
        ],
    )


_k2 = _make_k2(False)
_k2_final = _make_k2(True)


def kernel(edge_index, user_emb, item_emb):
    row = edge_index[0]
    col = edge_index[1]
    row = jnp.where(row >= NU, row + (PADB - NU), row)
    col = jnp.where(col >= NU, col + (PADB - NU), col)
    pad_e = E_PAD - E
    row3 = jnp.concatenate(
        [row, jnp.zeros((pad_e,), jnp.int32)]).reshape(EROWS, CHUNK)
    col3 = jnp.concatenate(
        [col, jnp.full((pad_e,), DUMMY, jnp.int32)]).reshape(EROWS, CHUNK)
    zeros = jnp.zeros((ROWS_PER_SUB, QF), jnp.float32)
    ones = jnp.ones((CHUNK, QF), jnp.float32)

    dis, y, s_acc = _k1(col3, user_emb, item_emb, zeros, ones)
    for _ in range(2):
        y, s_acc = _k2(row3, col3, y, dis, s_acc, zeros)
    user_f, item_f = _k2_final(row3, col3, y, dis, s_acc, zeros)
    return user_f, item_f


# speedup vs baseline: 1.6198x; 1.1169x over previous
import functools

import jax
import jax.numpy as jnp
from jax import lax
from jax.experimental import pallas as pl
from jax.experimental.pallas import tpu as pltpu
from jax.experimental.pallas import tpu_sc as plsc

NU = 25000
PADB = 25024
NACC = 2 * PADB
QF = 16
NQ = 4
NSUB = 16
NCORE = 2
CHUNK = 128
GROUP = 14
ITERS = 28
EROWS = NSUB * ITERS * GROUP
E = 800000
E_PAD = EROWS * CHUNK
SUB_EROWS = ITERS * GROUP
DUMMY = 25016
ROWS_PER_SUB = NACC // NSUB
BOUND_REAL = 3104
WB = ((0, 1000), (1000, 1000), (2000, 1000), (3000, 128))


def _sc_mesh():
    return plsc.VectorSubcoreMesh(core_axis_name="c", subcore_axis_name="s")


def _rsqrt16(d):
    i = lax.bitcast_convert_type(d, jnp.int32)
    i = jnp.int32(0x5F3759DF) - lax.shift_right_arithmetic(i, 1)
    y = lax.bitcast_convert_type(i, jnp.float32)
    for _ in range(3):
        y = y * (1.5 - 0.5 * d * y * y)
    return jnp.where(d > 0.0, y, 0.0)


def _zero_acc(s, zeros, acc):
    pltpu.sync_copy(zeros, acc.at[pl.ds(s * ROWS_PER_SUB, ROWS_PER_SUB)])


def _x_read(s, off, sz, user, item, zeros, xbuf, col_off):
    base = s * ROWS_PER_SUB + off
    csl = pl.ds(col_off, QF)
    if off < 3000:
        @pl.when(s < 8)
        def _():
            pltpu.sync_copy(user.at[pl.ds(base, sz), csl],
                            xbuf.at[pl.ds(0, sz)])

        @pl.when(s >= 8)
        def _():
            pltpu.sync_copy(item.at[pl.ds(base - PADB, sz), csl],
                            xbuf.at[pl.ds(0, sz)])
    else:
        clip = BOUND_REAL - off
        @pl.when(s < 7)
        def _():
            pltpu.sync_copy(user.at[pl.ds(base, sz), csl],
                            xbuf.at[pl.ds(0, sz)])

        @pl.when(s == 7)
        def _():
            pltpu.sync_copy(user.at[pl.ds(base, clip), csl],
                            xbuf.at[pl.ds(0, clip)])
            pltpu.sync_copy(zeros.at[pl.ds(0, sz - clip)],
                            xbuf.at[pl.ds(clip, sz - clip)])

        @pl.when((s >= 8) & (s < 15))
        def _():
            pltpu.sync_copy(item.at[pl.ds(base - PADB, sz), csl],
                            xbuf.at[pl.ds(0, sz)])

        @pl.when(s == 15)
        def _():
            pltpu.sync_copy(item.at[pl.ds(base - PADB, clip), csl],
                            xbuf.at[pl.ds(0, clip)])
            pltpu.sync_copy(zeros.at[pl.ds(0, sz - clip)],
                            xbuf.at[pl.ds(clip, sz - clip)])


def _out_write(s, off, sz, user_o, item_o, sbuf, col_off):
    base = s * ROWS_PER_SUB + off
    csl = pl.ds(col_off, QF)
    if off < 3000:
        @pl.when(s < 8)
        def _():
            pltpu.sync_copy(sbuf.at[pl.ds(0, sz)],
                            user_o.at[pl.ds(base, sz), csl])

        @pl.when(s >= 8)
        def _():
            pltpu.sync_copy(sbuf.at[pl.ds(0, sz)],
                            item_o.at[pl.ds(base - PADB, sz), csl])
    else:
        clip = BOUND_REAL - off
        @pl.when(s < 7)
        def _():
            pltpu.sync_copy(sbuf.at[pl.ds(0, sz)],
                            user_o.at[pl.ds(base, sz), csl])

        @pl.when(s == 7)
        def _():
            pltpu.sync_copy(sbuf.at[pl.ds(0, clip)],
                            user_o.at[pl.ds(base, clip), csl])

        @pl.when((s >= 8) & (s < 15))
        def _():
            pltpu.sync_copy(sbuf.at[pl.ds(0, sz)],
                            item_o.at[pl.ds(base - PADB, sz), csl])

        @pl.when(s == 15)
        def _():
            pltpu.sync_copy(sbuf.at[pl.ds(0, clip)],
                            item_o.at[pl.ds(base - PADB, clip), csl])


def _k1_body(col3, user, item, zeros, ones, dis_o, y_o, s_o,
             idx_c, ones_v, xbuf, dbuf, acc, sem_i, sem_s):
    c = lax.axis_index("c")
    s = lax.axis_index("s")
    pltpu.sync_copy(ones, ones_v)
    _zero_acc(s, zeros, acc)
    plsc.subcore_barrier()

    def deg_iter(g, car):
        base = s * SUB_EROWS + g * GROUP
        pltpu.async_copy(col3.at[pl.ds(base, GROUP)], idx_c, sem_i).wait()
        scs = [pltpu.async_copy(ones_v, acc.at[idx_c.at[j]], sem_s, add=True)
               for j in range(GROUP)]
        for x in scs:
            x.wait()
        return car

    lax.fori_loop(0, ITERS, deg_iter, 0)
    plsc.subcore_barrier()

    for off, sz in WB:
        absr = s * ROWS_PER_SUB + off
        pltpu.sync_copy(acc.at[pl.ds(absr, sz)], dbuf.at[pl.ds(0, sz)])

        def dis_row(i, car):
            for u in range(4):
                dbuf[i * 4 + u] = _rsqrt16(dbuf[i * 4 + u])
            return car

        lax.fori_loop(0, sz // 4, dis_row, 0)

        @pl.when(c == 0)
        def _():
            pltpu.sync_copy(dbuf.at[pl.ds(0, sz)], dis_o.at[pl.ds(absr, sz)])

        for p in range(2):
            q = 2 * c + p
            col_off = pl.multiple_of(q * QF, QF)
            _x_read(s, off, sz, user, item, zeros, xbuf, col_off)
            pltpu.sync_copy(xbuf.at[pl.ds(0, sz)],
                            s_o.at[q].at[pl.ds(absr, sz)])

            def y0_row(i, car):
                for u in range(4):
                    xbuf[i * 4 + u] = dbuf[i * 4 + u] * xbuf[i * 4 + u]
                return car

            lax.fori_loop(0, sz // 4, y0_row, 0)
            pltpu.sync_copy(xbuf.at[pl.ds(0, sz)],
                            y_o.at[q].at[pl.ds(absr, sz)])


def _k2_body(is_final, *refs):
    (row3, col3, y4, dis, s_in, zeros, out_a, out_b,
     idx_r, idx_c, rows_v, disb, sbuf, dbuf, acc,
     sem_i, sem_g, sem_s) = refs
    c = lax.axis_index("c")
    s = lax.axis_index("s")

    for p in range(2):
        q = 2 * c + p
        _zero_acc(s, zeros, acc)
        plsc.subcore_barrier()

        def edge_iter(g, car):
            base = s * SUB_EROWS + g * GROUP
            cp_r = pltpu.async_copy(row3.at[pl.ds(base, GROUP)], idx_r, sem_i)
            cp_c = pltpu.async_copy(col3.at[pl.ds(base, GROUP)], idx_c, sem_i)
            cp_r.wait()
            cp_c.wait()
            gs = [pltpu.async_copy(y4.at[q].at[idx_r.at[j]], rows_v.at[j],
                                   sem_g)
                  for j in range(GROUP)]
            for x in gs:
                x.wait()
            scs = [pltpu.async_copy(rows_v.at[j], acc.at[idx_c.at[j]], sem_s,
                                    add=True)
                   for j in range(GROUP)]
            for x in scs:
                x.wait()
            return car

        lax.fori_loop(0, ITERS, edge_iter, 0)
        plsc.subcore_barrier()

        for off, sz in WB:
            absr = s * ROWS_PER_SUB + off
            pltpu.sync_copy(acc.at[pl.ds(absr, sz)], dbuf.at[pl.ds(0, sz)])
            pltpu.sync_copy(dis.at[pl.ds(absr, sz)], disb.at[pl.ds(0, sz)])
            pltpu.sync_copy(s_in.at[q].at[pl.ds(absr, sz)],
                            sbuf.at[pl.ds(0, sz)])

            def wb_row(i, car):
                for u in range(4):
                    k = i * 4 + u
                    di = disb[k]
                    x = di * dbuf[k]
                    sn = sbuf[k] + x
                    if is_final:
                        sbuf[k] = sn * 0.25
                    else:
                        sbuf[k] = sn
                        dbuf[k] = di * x
                return car

            lax.fori_loop(0, sz // 4, wb_row, 0)

            if is_final:
                col_off = pl.multiple_of(q * QF, QF)
                _out_write(s, off, sz, out_a, out_b, sbuf, col_off)
            else:
                pltpu.sync_copy(sbuf.at[pl.ds(0, sz)],
                                out_b.at[q].at[pl.ds(absr, sz)])
                pltpu.sync_copy(dbuf.at[pl.ds(0, sz)],
                                out_a.at[q].at[pl.ds(absr, sz)])
        plsc.subcore_barrier()


_QSHAPE = jax.ShapeDtypeStruct((NQ, NACC, QF), jnp.float32)

_k1 = pl.kernel(
    _k1_body,
    out_type=(
        jax.ShapeDtypeStruct((NACC, QF), jnp.float32),
        _QSHAPE,
        _QSHAPE,
    ),
    mesh=_sc_mesh(),
    compiler_params=pltpu.CompilerParams(use_tc_tiling_on_sc=False),
    scratch_types=[
        pltpu.VMEM((GROUP, CHUNK), jnp.int32),
        pltpu.VMEM((CHUNK, QF), jnp.float32),
        pltpu.VMEM((1000, QF), jnp.float32),
        pltpu.VMEM((1000, QF), jnp.float32),
        pltpu.VMEM_SHARED((NACC, QF), jnp.float32),
        pltpu.SemaphoreType.DMA,
        pltpu.SemaphoreType.DMA,
    ],
)


def _make_k2(is_final):
    if is_final:
        out_type = (
            jax.ShapeDtypeStruct((NU, 4 * QF), jnp.float32),
            jax.ShapeDtypeStruct((NU, 4 * QF), jnp.float32),
        )
    else:
        out_type = (_QSHAPE, _QSHAPE)
    return pl.kernel(
        functools.partial(_k2_body, is_final),
        out_type=out_type,
        mesh=_sc_mesh(),
        compiler_params=pltpu.CompilerParams(use_tc_tiling_on_sc=False),
        scratch_types=[
            pltpu.VMEM((GROUP, CHUNK), jnp.int32),
            pltpu.VMEM((GROUP, CHUNK), jnp.int32),
            pltpu.VMEM((GROUP, CHUNK, QF), jnp.float32),
            pltpu.VMEM((1000, QF), jnp.float32),
            pltpu.VMEM((1000, QF), jnp.float32),
            pltpu.VMEM((1000, QF), jnp.float32),
            pltpu.VMEM_SHARED((NACC, QF), jnp.float32),
            pltpu.SemaphoreType.DMA,
            pltpu.SemaphoreType.DMA,
            pltpu.SemaphoreType.DMA,
        ],
    )


_k2 = _make_k2(False)
_k2_final = _make_k2(True)


def kernel(edge_index, user_emb, item_emb):
    row = edge_index[0]
    col = edge_index[1]
    row = jnp.where(row >= NU, row + (PADB - NU), row)
    col = jnp.where(col >= NU, col + (PADB - NU), col)
    pad_e = E_PAD - E
    row3 = jnp.concatenate(
        [row, jnp.zeros((pad_e,), jnp.int32)]).reshape(EROWS, CHUNK)
    col3 = jnp.concatenate(
        [col, jnp.full((pad_e,), DUMMY, jnp.int32)]).reshape(EROWS, CHUNK)
    zeros = jnp.zeros((ROWS_PER_SUB, QF), jnp.float32)
    ones = jnp.ones((CHUNK, QF), jnp.float32)

    dis, y, s_acc = _k1(col3, user_emb, item_emb, zeros, ones)
    for _ in range(2):
        y, s_acc = _k2(row3, col3, y, dis, s_acc, zeros)
    user_f, item_f = _k2_final(row3, col3, y, dis, s_acc, zeros)
    return user_f, item_f


# speedup vs baseline: 1.7155x; 1.0590x over previous
import functools

import jax
import jax.numpy as jnp
from jax import lax
from jax.experimental import pallas as pl
from jax.experimental.pallas import tpu as pltpu
from jax.experimental.pallas import tpu_sc as plsc

NU = 25000
PADB = 25024
NACC = 2 * PADB
QF = 16
NQ = 4
NSUB = 16
NCORE = 2
CHUNK = 128
GROUP = 14
ITERS = 28
EROWS = NSUB * ITERS * GROUP
E = 800000
E_PAD = EROWS * CHUNK
SUB_EROWS = ITERS * GROUP
DUMMY = 25016
ROWS_PER_SUB = NACC // NSUB
BOUND_REAL = 3104
WB = ((0, 512), (512, 512), (1024, 512), (1536, 512),
      (2048, 512), (2560, 512), (3072, 56))
WB_LAST = 3072


def _sc_mesh():
    return plsc.VectorSubcoreMesh(core_axis_name="c", subcore_axis_name="s")


def _rsqrt16(d):
    i = lax.bitcast_convert_type(d, jnp.int32)
    i = jnp.int32(0x5F3759DF) - lax.shift_right_arithmetic(i, 1)
    y = lax.bitcast_convert_type(i, jnp.float32)
    for _ in range(3):
        y = y * (1.5 - 0.5 * d * y * y)
    return jnp.where(d > 0.0, y, 0.0)


def _zero_acc(s, zeros, acc):
    pltpu.sync_copy(zeros, acc.at[pl.ds(s * ROWS_PER_SUB, ROWS_PER_SUB)])


def _x_read(s, off, sz, user, item, zeros, xbuf, col_off):
    base = s * ROWS_PER_SUB + off
    csl = pl.ds(col_off, QF)
    if off < WB_LAST:
        @pl.when(s < 8)
        def _():
            pltpu.sync_copy(user.at[pl.ds(base, sz), csl],
                            xbuf.at[pl.ds(0, sz)])

        @pl.when(s >= 8)
        def _():
            pltpu.sync_copy(item.at[pl.ds(base - PADB, sz), csl],
                            xbuf.at[pl.ds(0, sz)])
    else:
        clip = BOUND_REAL - off
        @pl.when(s < 7)
        def _():
            pltpu.sync_copy(user.at[pl.ds(base, sz), csl],
                            xbuf.at[pl.ds(0, sz)])

        @pl.when(s == 7)
        def _():
            pltpu.sync_copy(user.at[pl.ds(base, clip), csl],
                            xbuf.at[pl.ds(0, clip)])
            pltpu.sync_copy(zeros.at[pl.ds(0, sz - clip)],
                            xbuf.at[pl.ds(clip, sz - clip)])

        @pl.when((s >= 8) & (s < 15))
        def _():
            pltpu.sync_copy(item.at[pl.ds(base - PADB, sz), csl],
                            xbuf.at[pl.ds(0, sz)])

        @pl.when(s == 15)
        def _():
            pltpu.sync_copy(item.at[pl.ds(base - PADB, clip), csl],
                            xbuf.at[pl.ds(0, clip)])
            pltpu.sync_copy(zeros.at[pl.ds(0, sz - clip)],
                            xbuf.at[pl.ds(clip, sz - clip)])


def _out_write(s, off, sz, user_o, item_o, sbuf, col_off):
    base = s * ROWS_PER_SUB + off
    csl = pl.ds(col_off, QF)
    if off < WB_LAST:
        @pl.when(s < 8)
        def _():
            pltpu.sync_copy(sbuf.at[pl.ds(0, sz)],
                            user_o.at[pl.ds(base, sz), csl])

        @pl.when(s >= 8)
        def _():
            pltpu.sync_copy(sbuf.at[pl.ds(0, sz)],
                            item_o.at[pl.ds(base - PADB, sz), csl])
    else:
        clip = BOUND_REAL - off
        @pl.when(s < 7)
        def _():
            pltpu.sync_copy(sbuf.at[pl.ds(0, sz)],
                            user_o.at[pl.ds(base, sz), csl])

        @pl.when(s == 7)
        def _():
            pltpu.sync_copy(sbuf.at[pl.ds(0, clip)],
                            user_o.at[pl.ds(base, clip), csl])

        @pl.when((s >= 8) & (s < 15))
        def _():
            pltpu.sync_copy(sbuf.at[pl.ds(0, sz)],
                            item_o.at[pl.ds(base - PADB, sz), csl])

        @pl.when(s == 15)
        def _():
            pltpu.sync_copy(sbuf.at[pl.ds(0, clip)],
                            item_o.at[pl.ds(base - PADB, clip), csl])


def _k1_body(col3, user, item, zeros, ones, dis_o, y_o, s_o,
             idx_c, ones_v, xbuf, dbuf, acc, sem_i, sem_s):
    c = lax.axis_index("c")
    s = lax.axis_index("s")
    pltpu.sync_copy(ones, ones_v)
    _zero_acc(s, zeros, acc)
    plsc.subcore_barrier()

    def cload(g, slot):
        base = s * SUB_EROWS + g * GROUP
        return pltpu.async_copy(col3.at[pl.ds(base, GROUP)], idx_c.at[slot],
                                sem_i)

    def deg_scat(slot):
        scs = [pltpu.async_copy(ones_v, acc.at[idx_c.at[slot].at[j]], sem_s,
                                add=True)
               for j in range(GROUP)]
        for x in scs:
            x.wait()

    cload(0, 0).wait()

    def deg_iter(h, car):
        a = 2 * h
        cp1 = cload(a + 1, 1)
        deg_scat(0)
        cp2 = cload(jnp.where(a + 2 >= ITERS, 0, a + 2), 0)
        cp1.wait()
        deg_scat(1)
        cp2.wait()
        return car

    lax.fori_loop(0, ITERS // 2, deg_iter, 0)
    plsc.subcore_barrier()

    for off, sz in WB:
        absr = s * ROWS_PER_SUB + off
        pltpu.sync_copy(acc.at[pl.ds(absr, sz)], dbuf.at[pl.ds(0, sz)])

        def dis_row(i, car):
            for u in range(4):
                dbuf[i * 4 + u] = _rsqrt16(dbuf[i * 4 + u])
            return car

        lax.fori_loop(0, sz // 4, dis_row, 0)

        @pl.when(c == 0)
        def _():
            pltpu.sync_copy(dbuf.at[pl.ds(0, sz)], dis_o.at[pl.ds(absr, sz)])

        for p in range(2):
            q = 2 * c + p
            col_off = pl.multiple_of(q * QF, QF)
            _x_read(s, off, sz, user, item, zeros, xbuf, col_off)
            pltpu.sync_copy(xbuf.at[pl.ds(0, sz)],
                            s_o.at[q].at[pl.ds(absr, sz)])

            def y0_row(i, car):
                for u in range(4):
                    xbuf[i * 4 + u] = dbuf[i * 4 + u] * xbuf[i * 4 + u]
                return car

            lax.fori_loop(0, sz // 4, y0_row, 0)
            pltpu.sync_copy(xbuf.at[pl.ds(0, sz)],
                            y_o.at[q].at[pl.ds(absr, sz)])


def _k2_body(is_final, *refs):
    (row3, col3, y4, dis, s_in, zeros, out_a, out_b,
     idx_r, idx_c, rows_v, disb, sbuf, dbuf, acc,
     sem_i, sem_g, sem_s) = refs
    c = lax.axis_index("c")
    s = lax.axis_index("s")

    for p in range(2):
        q = 2 * c + p
        _zero_acc(s, zeros, acc)
        plsc.subcore_barrier()

        def rload(g, slot):
            base = s * SUB_EROWS + g * GROUP
            return pltpu.async_copy(row3.at[pl.ds(base, GROUP)],
                                    idx_r.at[slot], sem_i)

        def cload(g, slot):
            base = s * SUB_EROWS + g * GROUP
            return pltpu.async_copy(col3.at[pl.ds(base, GROUP)],
                                    idx_c.at[slot], sem_i)

        def gs_pass(slot):
            gs = [pltpu.async_copy(y4.at[q].at[idx_r.at[slot].at[j]],
                                   rows_v.at[j], sem_g)
                  for j in range(GROUP)]
            for x in gs:
                x.wait()
            scs = [pltpu.async_copy(rows_v.at[j],
                                    acc.at[idx_c.at[slot].at[j]], sem_s,
                                    add=True)
                   for j in range(GROUP)]
            for x in scs:
                x.wait()

        rload(0, 0).wait()
        cload(0, 0).wait()

        def edge_iter(h, car):
            a = 2 * h
            cp1r = rload(a + 1, 1)
            cp1c = cload(a + 1, 1)
            gs_pass(0)
            nxt = jnp.where(a + 2 >= ITERS, 0, a + 2)
            cp2r = rload(nxt, 0)
            cp2c = cload(nxt, 0)
            cp1r.wait()
            cp1c.wait()
            gs_pass(1)
            cp2r.wait()
            cp2c.wait()
            return car

        lax.fori_loop(0, ITERS // 2, edge_iter, 0)
        plsc.subcore_barrier()

        for off, sz in WB:
            absr = s * ROWS_PER_SUB + off
            pltpu.sync_copy(acc.at[pl.ds(absr, sz)], dbuf.at[pl.ds(0, sz)])
            pltpu.sync_copy(dis.at[pl.ds(absr, sz)], disb.at[pl.ds(0, sz)])
            pltpu.sync_copy(s_in.at[q].at[pl.ds(absr, sz)],
                            sbuf.at[pl.ds(0, sz)])

            def wb_row(i, car):
                for u in range(4):
                    k = i * 4 + u
                    di = disb[k]
                    x = di * dbuf[k]
                    sn = sbuf[k] + x
                    if is_final:
                        sbuf[k] = sn * 0.25
                    else:
                        sbuf[k] = sn
                        dbuf[k] = di * x
                return car

            lax.fori_loop(0, sz // 4, wb_row, 0)

            if is_final:
                col_off = pl.multiple_of(q * QF, QF)
                _out_write(s, off, sz, out_a, out_b, sbuf, col_off)
            else:
                pltpu.sync_copy(sbuf.at[pl.ds(0, sz)],
                                out_b.at[q].at[pl.ds(absr, sz)])
                pltpu.sync_copy(dbuf.at[pl.ds(0, sz)],
                                out_a.at[q].at[pl.ds(absr, sz)])
        plsc.subcore_barrier()


_QSHAPE = jax.ShapeDtypeStruct((NQ, NACC, QF), jnp.float32)

_k1 = pl.kernel(
    _k1_body,
    out_type=(
        jax.ShapeDtypeStruct((NACC, QF), jnp.float32),
        _QSHAPE,
        _QSHAPE,
    ),
    mesh=_sc_mesh(),
    compiler_params=pltpu.CompilerParams(use_tc_tiling_on_sc=False),
    scratch_types=[
        pltpu.VMEM((2, GROUP, CHUNK), jnp.int32),
        pltpu.VMEM((CHUNK, QF), jnp.float32),
        pltpu.VMEM((512, QF), jnp.float32),
        pltpu.VMEM((512, QF), jnp.float32),
        pltpu.VMEM_SHARED((NACC, QF), jnp.float32),
        pltpu.SemaphoreType.DMA,
        pltpu.SemaphoreType.DMA,
    ],
)


def _make_k2(is_final):
    if is_final:
        out_type = (
            jax.ShapeDtypeStruct((NU, 4 * QF), jnp.float32),
            jax.ShapeDtypeStruct((NU, 4 * QF), jnp.float32),
        )
    else:
        out_type = (_QSHAPE, _QSHAPE)
    return pl.kernel(
        functools.partial(_k2_body, is_final),
        out_type=out_type,
        mesh=_sc_mesh(),
        compiler_params=pltpu.CompilerParams(use_tc_tiling_on_sc=False),
        scratch_types=[
            pltpu.VMEM((2, GROUP, CHUNK), jnp.int32),
            pltpu.VMEM((2, GROUP, CHUNK), jnp.int32),
            pltpu.VMEM((GROUP, CHUNK, QF), jnp.float32),
            pltpu.VMEM((512, QF), jnp.float32),
            pltpu.VMEM((512, QF), jnp.float32),
            pltpu.VMEM((512, QF), jnp.float32),
            pltpu.VMEM_SHARED((NACC, QF), jnp.float32),
            pltpu.SemaphoreType.DMA,
            pltpu.SemaphoreType.DMA,
            pltpu.SemaphoreType.DMA,
        ],
    )


_k2 = _make_k2(False)
_k2_final = _make_k2(True)


def kernel(edge_index, user_emb, item_emb):
    row = edge_index[0]
    col = edge_index[1]
    row = jnp.where(row >= NU, row + (PADB - NU), row)
    col = jnp.where(col >= NU, col + (PADB - NU), col)
    pad_e = E_PAD - E
    row3 = jnp.concatenate(
        [row, jnp.zeros((pad_e,), jnp.int32)]).reshape(EROWS, CHUNK)
    col3 = jnp.concatenate(
        [col, jnp.full((pad_e,), DUMMY, jnp.int32)]).reshape(EROWS, CHUNK)
    zeros = jnp.zeros((ROWS_PER_SUB, QF), jnp.float32)
    ones = jnp.ones((CHUNK, QF), jnp.float32)

    dis, y, s_acc = _k1(col3, user_emb, item_emb, zeros, ones)
    for _ in range(2):
        y, s_acc = _k2(row3, col3, y, dis, s_acc, zeros)
    user_f, item_f = _k2_final(row3, col3, y, dis, s_acc, zeros)
    return user_f, item_f
